# Initial kernel scaffold; baseline (speedup 1.0000x reference)
import jax
import jax.numpy as jnp
from jax.experimental import pallas as pl


def kernel(v, table):
    raise NotImplementedError("write your pallas kernel here")


# speedup vs baseline: 3.1909x; 3.1909x over previous
import functools

import jax
import jax.numpy as jnp
from jax import lax
from jax.experimental import pallas as pl
from jax.experimental.pallas import tpu as pltpu
from jax.experimental.pallas import tpu_sc as plsc

B = 4
E = 32
N = 64 * 64 * 64
TOTAL = B * N
NW = 32
C = 1024
NCH = TOTAL // (NW * C)
G = 128
NG = C // G
L = 16


def _sc_body(v_hbm, table_hbm, out_hbm, idx_v, rows_v, out_t, gsem, wsem):
    cid = lax.axis_index("c")
    sid = lax.axis_index("s")
    wid = sid * 2 + cid

    def chunk_body(ch, carry):
        cix = wid * NCH + ch
        base = cix * C
        pltpu.sync_copy(v_hbm.at[pl.ds(cix * NG, NG)], idx_v)
        descs = [
            pltpu.async_copy(
                table_hbm.at[idx_v.at[j]], rows_v.at[pl.ds(j * G, G)], gsem
            )
            for j in range(NG)
        ]
        for d in descs:
            d.wait()
        def tr_body(i, carry2):
            rix = i * L + lax.iota(jnp.int32, L)
            for e in range(E):
                eix = jnp.full((L,), e, jnp.int32)
                out_t[e, pl.ds(i * L, L)] = plsc.load_gather(rows_v, [rix, eix])
            return carry2

        lax.fori_loop(0, C // L, tr_body, 0)
        b = base >> 18
        col = base & (N - 1)
        wdescs = [
            pltpu.async_copy(
                out_t.at[e],
                out_hbm.at[pl.ds(pl.multiple_of((b * E + e) * N + col, 8), C)],
                wsem,
            )
            for e in range(E)
        ]
        for d in wdescs:
            d.wait()
        return carry

    lax.fori_loop(0, NCH, chunk_body, 0)


@functools.partial(jax.jit, static_argnames=())
def kernel(v, table):
    mesh = plsc.VectorSubcoreMesh(core_axis_name="c", subcore_axis_name="s")
    k = functools.partial(
        pl.kernel,
        mesh=mesh,
        compiler_params=pltpu.CompilerParams(
            use_tc_tiling_on_sc=False, needs_layout_passes=False
        ),
        out_type=jax.ShapeDtypeStruct((B * E * N,), jnp.float32),
        scratch_types=[
            pltpu.VMEM((NG, G), jnp.int32),
            pltpu.VMEM((C, E), jnp.float32),
            pltpu.VMEM((E, C), jnp.float32),
            pltpu.SemaphoreType.DMA,
            pltpu.SemaphoreType.DMA,
        ],
    )(_sc_body)
    v2d = v.reshape(TOTAL // G, G)
    out = k(v2d, table)
    return out.reshape(B, E, 64, 64, 64)
